# Initial kernel scaffold; baseline (speedup 1.0000x reference)
import jax
import jax.numpy as jnp
from jax.experimental import pallas as pl


def kernel(input_ids, emb_weight, proj_weight):
    raise NotImplementedError("write your pallas kernel here")


# speedup vs baseline: 1.5023x; 1.5023x over previous
import functools

import jax
import jax.numpy as jnp
from jax import lax
from jax.experimental import pallas as pl
from jax.experimental.pallas import tpu as pltpu
from jax.experimental.pallas import tpu_sc as plsc

NUM_BUCKETS = 100000
MODEL_DIM = 2048
INNER_DIM = 128
MULT_PREV = 36313
MULT_CUR = 27191

_NC = 2
_NS = 16
_NW = _NC * _NS


def _gather_sc(ids, prev, emb_weight):
    n = ids.shape[0]
    per_w = n // _NW
    n_vec = per_w // 16
    n_dma = per_w // 128

    mesh = plsc.VectorSubcoreMesh(core_axis_name="c", subcore_axis_name="s")

    @functools.partial(
        pl.kernel,
        mesh=mesh,
        out_type=jax.ShapeDtypeStruct((n, INNER_DIM), jnp.float32),
        scratch_types=[
            pltpu.VMEM((per_w,), jnp.int32),
            pltpu.VMEM((per_w,), jnp.int32),
            pltpu.VMEM((per_w,), jnp.int32),
            pltpu.VMEM((per_w, INNER_DIM), jnp.float32),
            pltpu.SemaphoreType.DMA,
        ],
    )
    def gather_kernel(ids_hbm, prev_hbm, table_hbm, out_hbm, ids_v, prev_v, idx_v, rows_v, sem):
        wid = lax.axis_index("s") * _NC + lax.axis_index("c")
        base = wid * per_w
        pltpu.sync_copy(ids_hbm.at[pl.ds(base, per_w)], ids_v)
        pltpu.sync_copy(prev_hbm.at[pl.ds(base, per_w)], prev_v)

        @pl.loop(jnp.int32(0), jnp.int32(n_vec))
        def hash_body(i):
            off = i * jnp.int32(16)
            c = ids_v[pl.ds(off, 16)].astype(jnp.uint32)
            p = prev_v[pl.ds(off, 16)].astype(jnp.uint32)
            s = p * jnp.uint32(MULT_PREV) + c * jnp.uint32(MULT_CUR)
            q = (s.astype(jnp.float32) * jnp.float32(1.0 / NUM_BUCKETS)).astype(jnp.uint32)
            r = s - q * jnp.uint32(NUM_BUCKETS)
            r = jnp.where(r > jnp.uint32(3_000_000_000), r + jnp.uint32(NUM_BUCKETS), r)
            r = jnp.where(r >= jnp.uint32(NUM_BUCKETS), r - jnp.uint32(NUM_BUCKETS), r)
            idx_v[pl.ds(off, 16)] = r.astype(jnp.int32)

        copies = [
            pltpu.async_copy(
                table_hbm.at[idx_v.at[pl.ds(j * 128, 128)]],
                rows_v.at[pl.ds(j * 128, 128)],
                sem,
            )
            for j in range(n_dma)
        ]
        for cp in copies:
            cp.wait()
        pltpu.sync_copy(rows_v, out_hbm.at[pl.ds(base, per_w)])

    return gather_kernel(ids, prev, emb_weight)


def _matmul_tc(e, proj_weight, block_m=512):
    n = e.shape[0]

    def mm_body(e_ref, w_ref, o_ref):
        o_ref[...] = lax.dot_general(
            e_ref[...], w_ref[...],
            (((1,), (1,)), ((), ())),
            preferred_element_type=jnp.float32,
        )

    return pl.pallas_call(
        mm_body,
        grid=(n // block_m,),
        in_specs=[
            pl.BlockSpec((block_m, INNER_DIM), lambda i: (i, jnp.int32(0))),
            pl.BlockSpec((MODEL_DIM, INNER_DIM), lambda i: (jnp.int32(0), jnp.int32(0))),
        ],
        out_specs=pl.BlockSpec((block_m, MODEL_DIM), lambda i: (i, jnp.int32(0))),
        out_shape=jax.ShapeDtypeStruct((n, MODEL_DIM), jnp.float32),
    )(e, proj_weight)


def kernel(input_ids, emb_weight, proj_weight):
    b, s = input_ids.shape
    ids32 = input_ids.astype(jnp.int32)
    prev32 = jnp.pad(ids32[:, :-1], ((0, 0), (1, 0)))
    e = _gather_sc(ids32.reshape(-1), prev32.reshape(-1), emb_weight)
    out = _matmul_tc(e, proj_weight)
    return out.reshape(b, s, MODEL_DIM)


# speedup vs baseline: 1.5050x; 1.0018x over previous
import functools

import jax
import jax.numpy as jnp
from jax import lax
from jax.experimental import pallas as pl
from jax.experimental.pallas import tpu as pltpu
from jax.experimental.pallas import tpu_sc as plsc

NUM_BUCKETS = 100000
MODEL_DIM = 2048
INNER_DIM = 128
MULT_PREV = 36313
MULT_CUR = 27191

_NC = 2
_NS = 16
_NW = _NC * _NS


def _gather_sc(ids, prev, emb_weight):
    n = ids.shape[0]
    per_w = n // _NW
    n_vec = per_w // 16
    n_dma = per_w // 128

    mesh = plsc.VectorSubcoreMesh(core_axis_name="c", subcore_axis_name="s")

    @functools.partial(
        pl.kernel,
        mesh=mesh,
        out_type=jax.ShapeDtypeStruct((n, INNER_DIM), jnp.float32),
        scratch_types=[
            pltpu.VMEM((per_w,), jnp.int32),
            pltpu.VMEM((per_w,), jnp.int32),
            pltpu.VMEM((per_w,), jnp.int32),
            pltpu.VMEM((per_w, INNER_DIM), jnp.float32),
            pltpu.SemaphoreType.DMA,
        ],
    )
    def gather_kernel(ids_hbm, prev_hbm, table_hbm, out_hbm, ids_v, prev_v, idx_v, rows_v, sem):
        wid = lax.axis_index("s") * _NC + lax.axis_index("c")
        base = wid * per_w
        pltpu.sync_copy(ids_hbm.at[pl.ds(base, per_w)], ids_v)
        pltpu.sync_copy(prev_hbm.at[pl.ds(base, per_w)], prev_v)

        @pl.loop(jnp.int32(0), jnp.int32(n_vec))
        def hash_body(i):
            off = i * jnp.int32(16)
            c = ids_v[pl.ds(off, 16)].astype(jnp.uint32)
            p = prev_v[pl.ds(off, 16)].astype(jnp.uint32)
            s = p * jnp.uint32(MULT_PREV) + c * jnp.uint32(MULT_CUR)
            q = (s.astype(jnp.float32) * jnp.float32(1.0 / NUM_BUCKETS)).astype(jnp.uint32)
            r = s - q * jnp.uint32(NUM_BUCKETS)
            r = jnp.where(r > jnp.uint32(3_000_000_000), r + jnp.uint32(NUM_BUCKETS), r)
            r = jnp.where(r >= jnp.uint32(NUM_BUCKETS), r - jnp.uint32(NUM_BUCKETS), r)
            idx_v[pl.ds(off, 16)] = r.astype(jnp.int32)

        copies = [
            pltpu.async_copy(
                table_hbm.at[idx_v.at[pl.ds(j * 128, 128)]],
                rows_v.at[pl.ds(j * 128, 128)],
                sem,
            )
            for j in range(n_dma)
        ]
        for cp in copies:
            cp.wait()
        pltpu.sync_copy(rows_v, out_hbm.at[pl.ds(base, per_w)])

    return gather_kernel(ids, prev, emb_weight)


def _matmul_tc(e, proj_weight, block_m=512):
    n = e.shape[0]

    def mm_body(e_ref, w_ref, o_ref):
        o_ref[...] = lax.dot_general(
            e_ref[...].astype(jnp.bfloat16), w_ref[...].astype(jnp.bfloat16),
            (((1,), (1,)), ((), ())),
            preferred_element_type=jnp.float32,
        )

    return pl.pallas_call(
        mm_body,
        grid=(n // block_m,),
        in_specs=[
            pl.BlockSpec((block_m, INNER_DIM), lambda i: (i, jnp.int32(0))),
            pl.BlockSpec((MODEL_DIM, INNER_DIM), lambda i: (jnp.int32(0), jnp.int32(0))),
        ],
        out_specs=pl.BlockSpec((block_m, MODEL_DIM), lambda i: (i, jnp.int32(0))),
        out_shape=jax.ShapeDtypeStruct((n, MODEL_DIM), jnp.float32),
    )(e, proj_weight)


def kernel(input_ids, emb_weight, proj_weight):
    b, s = input_ids.shape
    ids32 = input_ids.astype(jnp.int32)
    prev32 = jnp.pad(ids32[:, :-1], ((0, 0), (1, 0)))
    e = _gather_sc(ids32.reshape(-1), prev32.reshape(-1), emb_weight)
    out = _matmul_tc(e, proj_weight)
    return out.reshape(b, s, MODEL_DIM)


# speedup vs baseline: 1.6324x; 1.0847x over previous
import functools

import jax
import jax.numpy as jnp
from jax import lax
from jax.experimental import pallas as pl
from jax.experimental.pallas import tpu as pltpu
from jax.experimental.pallas import tpu_sc as plsc

NUM_BUCKETS = 100000
MODEL_DIM = 2048
INNER_DIM = 128
MULT_PREV = 36313
MULT_CUR = 27191

_NC = 2
_NS = 16
_NW = _NC * _NS


def _gather_sc(ids, prev, emb_weight):
    n = ids.shape[0]
    per_w = n // _NW
    n_vec = per_w // 16
    n_dma = per_w // 128

    mesh = plsc.VectorSubcoreMesh(core_axis_name="c", subcore_axis_name="s")

    @functools.partial(
        pl.kernel,
        mesh=mesh,
        out_type=jax.ShapeDtypeStruct((n, INNER_DIM), jnp.float32),
        scratch_types=[
            pltpu.VMEM((per_w,), jnp.int32),
            pltpu.VMEM((per_w,), jnp.int32),
            pltpu.VMEM((per_w,), jnp.int32),
            pltpu.VMEM((per_w, INNER_DIM), jnp.float32),
            pltpu.SemaphoreType.DMA,
        ],
    )
    def gather_kernel(ids_hbm, prev_hbm, table_hbm, out_hbm, ids_v, prev_v, idx_v, rows_v, sem):
        wid = lax.axis_index("s") * _NC + lax.axis_index("c")
        base = wid * per_w
        pltpu.sync_copy(ids_hbm.at[pl.ds(base, per_w)], ids_v)
        pltpu.sync_copy(prev_hbm.at[pl.ds(base, per_w)], prev_v)

        @pl.loop(jnp.int32(0), jnp.int32(n_vec))
        def hash_body(i):
            off = i * jnp.int32(16)
            c = ids_v[pl.ds(off, 16)].astype(jnp.uint32)
            p = prev_v[pl.ds(off, 16)].astype(jnp.uint32)
            s = p * jnp.uint32(MULT_PREV) + c * jnp.uint32(MULT_CUR)
            q = (s.astype(jnp.float32) * jnp.float32(1.0 / NUM_BUCKETS)).astype(jnp.uint32)
            r = s - q * jnp.uint32(NUM_BUCKETS)
            r = jnp.where(r > jnp.uint32(3_000_000_000), r + jnp.uint32(NUM_BUCKETS), r)
            r = jnp.where(r >= jnp.uint32(NUM_BUCKETS), r - jnp.uint32(NUM_BUCKETS), r)
            idx_v[pl.ds(off, 16)] = r.astype(jnp.int32)

        copies = [
            pltpu.async_copy(
                table_hbm.at[idx_v.at[pl.ds(j * 128, 128)]],
                rows_v.at[pl.ds(j * 128, 128)],
                sem,
            )
            for j in range(n_dma)
        ]
        for cp in copies:
            cp.wait()
        pltpu.sync_copy(rows_v, out_hbm.at[pl.ds(base, per_w)])

    return gather_kernel(ids, prev, emb_weight)


def _matmul_tc(e, proj_weight, block_m=2048):
    n = e.shape[0]

    def mm_body(e_ref, w_ref, o_ref):
        o_ref[...] = lax.dot_general(
            e_ref[...].astype(jnp.bfloat16), w_ref[...].astype(jnp.bfloat16),
            (((1,), (1,)), ((), ())),
            preferred_element_type=jnp.float32,
        )

    return pl.pallas_call(
        mm_body,
        grid=(n // block_m,),
        in_specs=[
            pl.BlockSpec((block_m, INNER_DIM), lambda i: (i, jnp.int32(0))),
            pl.BlockSpec((MODEL_DIM, INNER_DIM), lambda i: (jnp.int32(0), jnp.int32(0))),
        ],
        out_specs=pl.BlockSpec((block_m, MODEL_DIM), lambda i: (i, jnp.int32(0))),
        out_shape=jax.ShapeDtypeStruct((n, MODEL_DIM), jnp.float32),
    )(e, proj_weight)


def kernel(input_ids, emb_weight, proj_weight):
    b, s = input_ids.shape
    ids32 = input_ids.astype(jnp.int32)
    prev32 = jnp.pad(ids32[:, :-1], ((0, 0), (1, 0)))
    e = _gather_sc(ids32.reshape(-1), prev32.reshape(-1), emb_weight)
    out = _matmul_tc(e, proj_weight)
    return out.reshape(b, s, MODEL_DIM)


# speedup vs baseline: 1.6495x; 1.0105x over previous
import functools

import jax
import jax.numpy as jnp
from jax import lax
from jax.experimental import pallas as pl
from jax.experimental.pallas import tpu as pltpu
from jax.experimental.pallas import tpu_sc as plsc

NUM_BUCKETS = 100000
MODEL_DIM = 2048
INNER_DIM = 128
MULT_PREV = 36313
MULT_CUR = 27191

_NC = 2
_NS = 16
_NW = _NC * _NS


def _gather_sc(ids, prev, emb_weight):
    n = ids.shape[0]
    per_w = n // _NW
    n_vec = per_w // 16
    n_dma = per_w // 128

    mesh = plsc.VectorSubcoreMesh(core_axis_name="c", subcore_axis_name="s")

    @functools.partial(
        pl.kernel,
        mesh=mesh,
        out_type=jax.ShapeDtypeStruct((n, INNER_DIM), jnp.float32),
        scratch_types=[
            pltpu.VMEM((per_w,), jnp.int32),
            pltpu.VMEM((per_w,), jnp.int32),
            pltpu.VMEM((per_w,), jnp.int32),
            pltpu.VMEM((per_w, INNER_DIM), jnp.float32),
            pltpu.SemaphoreType.DMA,
        ],
    )
    def gather_kernel(ids_hbm, prev_hbm, table_hbm, out_hbm, ids_v, prev_v, idx_v, rows_v, sem):
        wid = lax.axis_index("s") * _NC + lax.axis_index("c")
        base = wid * per_w
        pltpu.sync_copy(ids_hbm.at[pl.ds(base, per_w)], ids_v)
        pltpu.sync_copy(prev_hbm.at[pl.ds(base, per_w)], prev_v)

        @pl.loop(jnp.int32(0), jnp.int32(n_vec))
        def hash_body(i):
            off = i * jnp.int32(16)
            c = ids_v[pl.ds(off, 16)].astype(jnp.uint32)
            p = prev_v[pl.ds(off, 16)].astype(jnp.uint32)
            s = p * jnp.uint32(MULT_PREV) + c * jnp.uint32(MULT_CUR)
            q = (s.astype(jnp.float32) * jnp.float32(1.0 / NUM_BUCKETS)).astype(jnp.uint32)
            r = s - q * jnp.uint32(NUM_BUCKETS)
            r = jnp.where(r > jnp.uint32(3_000_000_000), r + jnp.uint32(NUM_BUCKETS), r)
            r = jnp.where(r >= jnp.uint32(NUM_BUCKETS), r - jnp.uint32(NUM_BUCKETS), r)
            idx_v[pl.ds(off, 16)] = r.astype(jnp.int32)

        copies = [
            pltpu.async_copy(
                table_hbm.at[idx_v.at[pl.ds(j * 128, 128)]],
                rows_v.at[pl.ds(j * 128, 128)],
                sem,
            )
            for j in range(n_dma)
        ]
        for cp in copies:
            cp.wait()
        pltpu.sync_copy(rows_v, out_hbm.at[pl.ds(base, per_w)])

    return gather_kernel(ids, prev, emb_weight)


def _matmul_tc(e, proj_weight, block_m=1024):
    n = e.shape[0]

    def mm_body(e_ref, w_ref, o_ref):
        o_ref[...] = lax.dot_general(
            e_ref[...].astype(jnp.bfloat16), w_ref[...].astype(jnp.bfloat16),
            (((1,), (1,)), ((), ())),
            preferred_element_type=jnp.float32,
        )

    return pl.pallas_call(
        mm_body,
        grid=(n // block_m,),
        in_specs=[
            pl.BlockSpec((block_m, INNER_DIM), lambda i: (i, jnp.int32(0))),
            pl.BlockSpec((MODEL_DIM, INNER_DIM), lambda i: (jnp.int32(0), jnp.int32(0))),
        ],
        out_specs=pl.BlockSpec((block_m, MODEL_DIM), lambda i: (i, jnp.int32(0))),
        out_shape=jax.ShapeDtypeStruct((n, MODEL_DIM), jnp.float32),
    )(e, proj_weight)


def kernel(input_ids, emb_weight, proj_weight):
    b, s = input_ids.shape
    ids32 = input_ids.astype(jnp.int32)
    prev32 = jnp.pad(ids32[:, :-1], ((0, 0), (1, 0)))
    e = _gather_sc(ids32.reshape(-1), prev32.reshape(-1), emb_weight)
    out = _matmul_tc(e, proj_weight)
    return out.reshape(b, s, MODEL_DIM)
